# baseline (speedup 1.0000x reference)
import functools

import jax
import jax.numpy as jnp
from jax import lax
from jax.experimental import pallas as pl
from jax.experimental.pallas import tpu as pltpu
from jax.experimental.pallas import tpu_sc as plsc

_N = 32768
_LANES = 16
_VREGS = _N // _LANES

_SC_ITERS = 8


def _sc_body(x_hbm, o_hbm, row_v, red_v, sem):
    core = lax.axis_index("c")
    sub = lax.axis_index("s")
    wid = sub * 2 + core
    rows = x_hbm.shape[0]
    rows_per = rows // 32

    def lanes_fold(vec, init, op):
        acc = init
        for i in range(_LANES):
            acc = op(acc, vec[i])
        return acc

    def do_row(r, carry):
        row = wid * rows_per + r
        pltpu.sync_copy(x_hbm.at[row], row_v)

        def maxbody(i, acc):
            return jnp.maximum(acc, row_v[pl.ds(i * _LANES, _LANES)])

        m16 = lax.fori_loop(0, _VREGS, maxbody,
                            jnp.full((_LANES,), -jnp.inf, jnp.float32))
        m = lanes_fold(m16, jnp.float32(-jnp.inf), jnp.maximum)
        tau0 = jnp.full((_LANES,), m, jnp.float32) - 1.0

        def michelot(_, tau):
            def sumbody(i, carry):
                s, c = carry
                d = row_v[pl.ds(i * _LANES, _LANES)] - tau
                s = s + jnp.maximum(d, 0.0)
                c = c + jnp.where(d > 0.0, 1.0, 0.0)
                return s, c

            z = jnp.zeros((_LANES,), jnp.float32)
            s16, c16 = lax.fori_loop(0, _VREGS, sumbody, (z, z))
            s = lanes_fold(s16, jnp.float32(0.0), jnp.add)
            c = lanes_fold(c16, jnp.float32(0.0), jnp.add)
            s16b = jnp.full((_LANES,), s - 1.0, jnp.float32)
            c16b = jnp.full((_LANES,), jnp.maximum(c, 1.0), jnp.float32)
            return tau + s16b / c16b

        tau = lax.fori_loop(0, _SC_ITERS, michelot, tau0)

        def outbody(i, _):
            sl = pl.ds(i * _LANES, _LANES)
            row_v[sl] = jnp.maximum(row_v[sl] - tau, 0.0)
            return 0

        lax.fori_loop(0, _VREGS, outbody, 0)
        pltpu.sync_copy(row_v, o_hbm.at[row])
        return carry

    lax.fori_loop(0, rows_per, do_row, 0)


def _sc_sparsemax(x):
    rows = x.shape[0]
    return pl.kernel(
        _sc_body,
        out_type=jax.ShapeDtypeStruct((rows, _N), jnp.float32),
        mesh=plsc.VectorSubcoreMesh(
            core_axis_name="c", subcore_axis_name="s",
            num_cores=2, num_subcores=16),
        scratch_types=[
            pltpu.VMEM((_N,), jnp.float32),
            pltpu.VMEM((_LANES,), jnp.float32),
            pltpu.SemaphoreType.DMA,
        ],
    )(x)


_ROWS_PER_BLOCK = 64
_MAX_ITERS = 16


def _tc_block(x_ref, o_ref):
    m = jnp.max(x_ref[...], axis=-1, keepdims=True)
    tau0 = m - 1.0

    def cond(carry):
        it, tau, prev = carry
        return jnp.logical_and(it < _MAX_ITERS, jnp.any(tau != prev))

    def body(carry):
        it, tau, _ = carry
        s = jnp.sum(jnp.maximum(x_ref[...] - tau, 0.0), axis=-1, keepdims=True)
        c = jnp.sum((x_ref[...] > tau).astype(jnp.float32), axis=-1, keepdims=True)
        new = tau + (s - 1.0) / jnp.maximum(c, 1.0)
        return it + 1, new, tau

    _, tau, _ = lax.while_loop(cond, body, (0, tau0, tau0 - 1.0))
    o_ref[...] = jnp.maximum(x_ref[...] - tau, 0.0)


def _tc_sparsemax(x):
    rows, n = x.shape
    r = min(_ROWS_PER_BLOCK, rows)
    return pl.pallas_call(
        _tc_block,
        out_shape=jax.ShapeDtypeStruct(x.shape, x.dtype),
        grid=(rows // r,),
        in_specs=[pl.BlockSpec((r, n), lambda i: (i, 0))],
        out_specs=pl.BlockSpec((r, n), lambda i: (i, 0)),
        compiler_params=pltpu.CompilerParams(
            dimension_semantics=("parallel",),
        ),
    )(x)


def kernel(x):
    return _sc_sparsemax(x)


# speedup vs baseline: 2.8101x; 2.8101x over previous
import functools

import jax
import jax.numpy as jnp
from jax import lax
from jax.experimental import pallas as pl
from jax.experimental.pallas import tpu as pltpu
from jax.experimental.pallas import tpu_sc as plsc

_N = 32768
_LANES = 16
_VREGS = _N // _LANES

_SC_ITERS = 8


def _sc_body(x_hbm, o_hbm, row_v, red_v, sem):
    core = lax.axis_index("c")
    sub = lax.axis_index("s")
    wid = sub * 2 + core
    rows = x_hbm.shape[0]
    rows_per = rows // 32

    def lanes_fold(vec, init, op):
        acc = init
        for i in range(_LANES):
            acc = op(acc, vec[i])
        return acc

    U = 8
    steps = _VREGS // U

    def do_row(r, carry):
        row = wid * rows_per + r
        pltpu.sync_copy(x_hbm.at[row], row_v)

        def maxbody(i, accs):
            base = i * _LANES * U
            return tuple(
                jnp.maximum(accs[u], row_v[pl.ds(base + u * _LANES, _LANES)])
                for u in range(U))

        neg = jnp.full((_LANES,), -jnp.inf, jnp.float32)
        maxs = lax.fori_loop(0, steps, maxbody, (neg,) * U)
        m16 = functools.reduce(jnp.maximum, maxs)
        m = lanes_fold(m16, jnp.float32(-jnp.inf), jnp.maximum)
        tau0 = jnp.full((_LANES,), m, jnp.float32) - 1.0

        def michelot(_, tau):
            def sumbody(i, carry):
                ss, cs = carry
                base = i * _LANES * U
                ss2, cs2 = [], []
                for u in range(U):
                    d = row_v[pl.ds(base + u * _LANES, _LANES)] - tau
                    ss2.append(ss[u] + jnp.maximum(d, 0.0))
                    cs2.append(cs[u] + jnp.where(d > 0.0, 1.0, 0.0))
                return tuple(ss2), tuple(cs2)

            z = jnp.zeros((_LANES,), jnp.float32)
            ss, cs = lax.fori_loop(0, steps, sumbody, ((z,) * U, (z,) * U))
            s = lanes_fold(functools.reduce(jnp.add, ss), jnp.float32(0.0), jnp.add)
            c = lanes_fold(functools.reduce(jnp.add, cs), jnp.float32(0.0), jnp.add)
            s16b = jnp.full((_LANES,), s - 1.0, jnp.float32)
            c16b = jnp.full((_LANES,), jnp.maximum(c, 1.0), jnp.float32)
            return tau + s16b / c16b

        tau = lax.fori_loop(0, _SC_ITERS, michelot, tau0)

        def outbody(i, _):
            base = i * _LANES * U
            for u in range(U):
                sl = pl.ds(base + u * _LANES, _LANES)
                row_v[sl] = jnp.maximum(row_v[sl] - tau, 0.0)
            return 0

        lax.fori_loop(0, steps, outbody, 0)
        pltpu.sync_copy(row_v, o_hbm.at[row])
        return carry

    lax.fori_loop(0, rows_per, do_row, 0)


def _sc_sparsemax(x):
    rows = x.shape[0]
    return pl.kernel(
        _sc_body,
        out_type=jax.ShapeDtypeStruct((rows, _N), jnp.float32),
        mesh=plsc.VectorSubcoreMesh(
            core_axis_name="c", subcore_axis_name="s",
            num_cores=2, num_subcores=16),
        scratch_types=[
            pltpu.VMEM((_N,), jnp.float32),
            pltpu.VMEM((_LANES,), jnp.float32),
            pltpu.SemaphoreType.DMA,
        ],
    )(x)


_ROWS_PER_BLOCK = 64
_MAX_ITERS = 16


def _tc_block(x_ref, o_ref):
    m = jnp.max(x_ref[...], axis=-1, keepdims=True)
    tau0 = m - 1.0

    def cond(carry):
        it, tau, prev = carry
        return jnp.logical_and(it < _MAX_ITERS, jnp.any(tau != prev))

    def body(carry):
        it, tau, _ = carry
        s = jnp.sum(jnp.maximum(x_ref[...] - tau, 0.0), axis=-1, keepdims=True)
        c = jnp.sum((x_ref[...] > tau).astype(jnp.float32), axis=-1, keepdims=True)
        new = tau + (s - 1.0) / jnp.maximum(c, 1.0)
        return it + 1, new, tau

    _, tau, _ = lax.while_loop(cond, body, (0, tau0, tau0 - 1.0))
    o_ref[...] = jnp.maximum(x_ref[...] - tau, 0.0)


def _tc_sparsemax(x):
    rows, n = x.shape
    r = next(b for b in (64, 48, 32, 16, 8, rows) if rows % b == 0)
    return pl.pallas_call(
        _tc_block,
        out_shape=jax.ShapeDtypeStruct(x.shape, x.dtype),
        grid=(rows // r,),
        in_specs=[pl.BlockSpec((r, n), lambda i: (i, 0))],
        out_specs=pl.BlockSpec((r, n), lambda i: (i, 0)),
        compiler_params=pltpu.CompilerParams(
            dimension_semantics=("parallel",),
        ),
    )(x)


def kernel(x):
    return _sc_sparsemax(x)


# speedup vs baseline: 4.7613x; 1.6944x over previous
import functools

import jax
import jax.numpy as jnp
from jax import lax
from jax.experimental import pallas as pl
from jax.experimental.pallas import tpu as pltpu
from jax.experimental.pallas import tpu_sc as plsc

_N = 32768
_LANES = 16
_VREGS = _N // _LANES

_SC_ITERS = 8


def _sc_body(x_hbm, o_hbm, row_v, red_v, sem):
    core = lax.axis_index("c")
    sub = lax.axis_index("s")
    wid = sub * 2 + core
    rows = x_hbm.shape[0]
    rows_per = rows // 32

    def lanes_fold(vec, init, op):
        acc = init
        for i in range(_LANES):
            acc = op(acc, vec[i])
        return acc

    U = 8
    steps = _VREGS // U

    def do_row(r, carry):
        row = wid * rows_per + r
        pltpu.sync_copy(x_hbm.at[row], row_v)

        def maxbody(i, accs):
            base = i * _LANES * U
            return tuple(
                jnp.maximum(accs[u], row_v[pl.ds(base + u * _LANES, _LANES)])
                for u in range(U))

        neg = jnp.full((_LANES,), -jnp.inf, jnp.float32)
        maxs = lax.fori_loop(0, steps, maxbody, (neg,) * U)
        m16 = functools.reduce(jnp.maximum, maxs)
        m = lanes_fold(m16, jnp.float32(-jnp.inf), jnp.maximum)
        tau0 = jnp.full((_LANES,), m, jnp.float32) - 1.0

        def michelot(_, tau):
            def sumbody(i, carry):
                ss, cs = carry
                base = i * _LANES * U
                ss2, cs2 = [], []
                for u in range(U):
                    d = row_v[pl.ds(base + u * _LANES, _LANES)] - tau
                    ss2.append(ss[u] + jnp.maximum(d, 0.0))
                    cs2.append(cs[u] + jnp.where(d > 0.0, 1.0, 0.0))
                return tuple(ss2), tuple(cs2)

            z = jnp.zeros((_LANES,), jnp.float32)
            ss, cs = lax.fori_loop(0, steps, sumbody, ((z,) * U, (z,) * U))
            s = lanes_fold(functools.reduce(jnp.add, ss), jnp.float32(0.0), jnp.add)
            c = lanes_fold(functools.reduce(jnp.add, cs), jnp.float32(0.0), jnp.add)
            s16b = jnp.full((_LANES,), s - 1.0, jnp.float32)
            c16b = jnp.full((_LANES,), jnp.maximum(c, 1.0), jnp.float32)
            return tau + s16b / c16b

        tau = lax.fori_loop(0, _SC_ITERS, michelot, tau0)

        def outbody(i, _):
            base = i * _LANES * U
            for u in range(U):
                sl = pl.ds(base + u * _LANES, _LANES)
                row_v[sl] = jnp.maximum(row_v[sl] - tau, 0.0)
            return 0

        lax.fori_loop(0, steps, outbody, 0)
        pltpu.sync_copy(row_v, o_hbm.at[row])
        return carry

    lax.fori_loop(0, rows_per, do_row, 0)


def _sc_sparsemax(x):
    rows = x.shape[0]
    return pl.kernel(
        _sc_body,
        out_type=jax.ShapeDtypeStruct((rows, _N), jnp.float32),
        mesh=plsc.VectorSubcoreMesh(
            core_axis_name="c", subcore_axis_name="s",
            num_cores=2, num_subcores=16),
        scratch_types=[
            pltpu.VMEM((_N,), jnp.float32),
            pltpu.VMEM((_LANES,), jnp.float32),
            pltpu.SemaphoreType.DMA,
        ],
    )(x)


_ROWS_PER_BLOCK = 64
_MAX_ITERS = 16


def _tc_block(x_ref, o_ref):
    m = jnp.max(x_ref[...], axis=-1, keepdims=True)
    tau0 = m - 1.0

    def cond(carry):
        it, tau, prev = carry
        return jnp.logical_and(it < _MAX_ITERS, jnp.any(tau != prev))

    def body(carry):
        it, tau, _ = carry
        d = x_ref[...] - tau
        s = jnp.sum(jnp.maximum(d, 0.0), axis=-1, keepdims=True)
        c = jnp.sum((d > 0.0).astype(jnp.float32), axis=-1, keepdims=True)
        new = tau + (s - 1.0) / jnp.maximum(c, 1.0)
        return it + 1, new, tau

    _, tau, _ = lax.while_loop(cond, body, (0, tau0, tau0 - 1.0))
    o_ref[...] = jnp.maximum(x_ref[...] - tau, 0.0)


def _tc_sparsemax(x):
    rows, n = x.shape
    r = next(b for b in (64, 48, 32, 16, 8, rows) if rows % b == 0)
    return pl.pallas_call(
        _tc_block,
        out_shape=jax.ShapeDtypeStruct(x.shape, x.dtype),
        grid=(rows // r,),
        in_specs=[pl.BlockSpec((r, n), lambda i: (i, 0))],
        out_specs=pl.BlockSpec((r, n), lambda i: (i, 0)),
        compiler_params=pltpu.CompilerParams(
            dimension_semantics=("parallel",),
        ),
    )(x)


_SC_SHARE = 32


def kernel(x):
    rows = x.shape[0]
    k = _SC_SHARE
    if not (0 < k < rows and k % 32 == 0):
        return _tc_sparsemax(x)
    tc = _tc_sparsemax(x[: rows - k])
    sc = _sc_sparsemax(x[rows - k:])
    return jnp.concatenate([tc, sc], axis=0)


# speedup vs baseline: 9.4422x; 1.9831x over previous
import functools

import jax
import jax.numpy as jnp
from jax import lax
from jax.experimental import pallas as pl
from jax.experimental.pallas import tpu as pltpu
from jax.experimental.pallas import tpu_sc as plsc

_N = 32768
_LANES = 16
_VREGS = _N // _LANES

_SC_ITERS = 8


def _sc_body(x_hbm, o_hbm, row_v, red_v, sem):
    core = lax.axis_index("c")
    sub = lax.axis_index("s")
    wid = sub * 2 + core
    rows = x_hbm.shape[0]
    rows_per = rows // 32

    def lanes_fold(vec, init, op):
        acc = init
        for i in range(_LANES):
            acc = op(acc, vec[i])
        return acc

    U = 8
    steps = _VREGS // U

    def do_row(r, carry):
        row = wid * rows_per + r
        pltpu.sync_copy(x_hbm.at[row], row_v)

        def maxbody(i, accs):
            base = i * _LANES * U
            return tuple(
                jnp.maximum(accs[u], row_v[pl.ds(base + u * _LANES, _LANES)])
                for u in range(U))

        neg = jnp.full((_LANES,), -jnp.inf, jnp.float32)
        maxs = lax.fori_loop(0, steps, maxbody, (neg,) * U)
        m16 = functools.reduce(jnp.maximum, maxs)
        m = lanes_fold(m16, jnp.float32(-jnp.inf), jnp.maximum)
        tau0 = jnp.full((_LANES,), m, jnp.float32) - 1.0

        def michelot(_, tau):
            def sumbody(i, carry):
                ss, cs = carry
                base = i * _LANES * U
                ss2, cs2 = [], []
                for u in range(U):
                    d = row_v[pl.ds(base + u * _LANES, _LANES)] - tau
                    ss2.append(ss[u] + jnp.maximum(d, 0.0))
                    cs2.append(cs[u] + jnp.where(d > 0.0, 1.0, 0.0))
                return tuple(ss2), tuple(cs2)

            z = jnp.zeros((_LANES,), jnp.float32)
            ss, cs = lax.fori_loop(0, steps, sumbody, ((z,) * U, (z,) * U))
            s = lanes_fold(functools.reduce(jnp.add, ss), jnp.float32(0.0), jnp.add)
            c = lanes_fold(functools.reduce(jnp.add, cs), jnp.float32(0.0), jnp.add)
            s16b = jnp.full((_LANES,), s - 1.0, jnp.float32)
            c16b = jnp.full((_LANES,), jnp.maximum(c, 1.0), jnp.float32)
            return tau + s16b / c16b

        tau = lax.fori_loop(0, _SC_ITERS, michelot, tau0)

        def outbody(i, _):
            base = i * _LANES * U
            for u in range(U):
                sl = pl.ds(base + u * _LANES, _LANES)
                row_v[sl] = jnp.maximum(row_v[sl] - tau, 0.0)
            return 0

        lax.fori_loop(0, steps, outbody, 0)
        pltpu.sync_copy(row_v, o_hbm.at[row])
        return carry

    lax.fori_loop(0, rows_per, do_row, 0)


def _sc_sparsemax(x):
    rows = x.shape[0]
    return pl.kernel(
        _sc_body,
        out_type=jax.ShapeDtypeStruct((rows, _N), jnp.float32),
        mesh=plsc.VectorSubcoreMesh(
            core_axis_name="c", subcore_axis_name="s",
            num_cores=2, num_subcores=16),
        scratch_types=[
            pltpu.VMEM((_N,), jnp.float32),
            pltpu.VMEM((_LANES,), jnp.float32),
            pltpu.SemaphoreType.DMA,
        ],
    )(x)


_ROWS_PER_BLOCK = 64
_MAX_ITERS = 16


def _tc_block(x_ref, o_ref):
    m = jnp.max(x_ref[...], axis=-1, keepdims=True)
    tau0 = m - 1.0

    def cond(carry):
        it, tau, prev = carry
        return jnp.logical_and(it < _MAX_ITERS, jnp.any(tau != prev))

    def body(carry):
        it, tau, _ = carry
        d = x_ref[...] - tau
        s = jnp.sum(jnp.maximum(d, 0.0), axis=-1, keepdims=True)
        c = jnp.sum((d > 0.0).astype(jnp.float32), axis=-1, keepdims=True)
        new = tau + (s - 1.0) / jnp.maximum(c, 1.0)
        return it + 1, new, tau

    _, tau, _ = lax.while_loop(cond, body, (0, tau0, tau0 - 1.0))
    o_ref[...] = jnp.maximum(x_ref[...] - tau, 0.0)


def _tc_sparsemax(x):
    rows, n = x.shape
    r = next(b for b in (64, 48, 32, 16, 8, rows) if rows % b == 0)
    return pl.pallas_call(
        _tc_block,
        out_shape=jax.ShapeDtypeStruct(x.shape, x.dtype),
        grid=(rows // r,),
        in_specs=[pl.BlockSpec((r, n), lambda i: (i, 0))],
        out_specs=pl.BlockSpec((r, n), lambda i: (i, 0)),
        compiler_params=pltpu.CompilerParams(
            dimension_semantics=("parallel",),
        ),
    )(x)


def kernel(x):
    return _tc_sparsemax(x)
